# baseline (speedup 1.0000x reference)
import functools

import jax
import jax.numpy as jnp
from jax import lax
from jax.experimental import pallas as pl
from jax.experimental.pallas import tpu as pltpu
from jax.experimental.pallas import tpu_sc as plsc

_N = 2048
_E = 8192
_G = 64
_W = 128
_BLK = 512
_KC = 16

_NW = 32
_EPW = _E // _NW
_EPT = _E // 16
_NH = _N // 2
_NPT = _NH // 16


def _sc_mesh():
    return plsc.VectorSubcoreMesh(core_axis_name="c", subcore_axis_name="s")


def _make_gather():
    @functools.partial(
        pl.kernel, mesh=_sc_mesh(),
        out_type=jax.ShapeDtypeStruct((_E, _W), jnp.float32),
        scratch_types=[
            pltpu.VMEM((2, 128), jnp.int32),
            pltpu.VMEM((_EPW, _W), jnp.float32),
            pltpu.SemaphoreType.DMA,
        ],
    )
    def g(x_hbm, src_hbm, out_hbm, idx_v, rows_v, sem):
        wid = lax.axis_index("s") * 2 + lax.axis_index("c")
        pltpu.sync_copy(src_hbm.at[pl.ds(wid * 2, 2)], idx_v)
        for j in range(2):
            pltpu.async_copy(x_hbm.at[idx_v.at[j]],
                             rows_v.at[pl.ds(j * 128, 128)], sem).wait()
        pltpu.sync_copy(rows_v, out_hbm.at[pl.ds(wid * _EPW, _EPW)])

    return g


def _make_scatter(with_gather):
    @functools.partial(
        pl.kernel, mesh=_sc_mesh(),
        out_type=(jax.ShapeDtypeStruct((_N, _W), jnp.float32),
                  jax.ShapeDtypeStruct((_E, _W), jnp.float32))
        if with_gather else jax.ShapeDtypeStruct((_N, _W), jnp.float32),
        scratch_types=[
            pltpu.VMEM_SHARED((_N, _W), jnp.float32),
            pltpu.VMEM((_EPT, _W), jnp.float32),
            pltpu.VMEM((4, 128), jnp.int32),
            pltpu.SemaphoreType.DMA,
        ],
    )
    def s(*refs):
        if with_gather:
            msg_hbm, dst_hbm, root_hbm, src_hbm, out_hbm, xs_hbm, acc, buf, idx_v, sem = refs
        else:
            msg_hbm, dst_hbm, root_hbm, out_hbm, acc, buf, idx_v, sem = refs
        c = lax.axis_index("c")
        sid = lax.axis_index("s")
        wid = sid * 2 + c
        npc = _N // 16
        pltpu.sync_copy(root_hbm.at[pl.ds(sid * npc, npc)],
                        buf.at[pl.ds(0, npc)])
        pltpu.sync_copy(buf.at[pl.ds(0, npc)], acc.at[pl.ds(sid * npc, npc)])
        plsc.subcore_barrier()
        pltpu.sync_copy(dst_hbm.at[pl.ds(sid * 4, 4)], idx_v)
        pltpu.sync_copy(msg_hbm.at[pl.ds(sid * _EPT, _EPT)], buf)
        for j in range(4):
            pltpu.sync_copy(buf.at[pl.ds(j * 128, 128)], acc.at[idx_v.at[j]],
                            add=True)
        plsc.subcore_barrier()
        nw = _N // 32
        pltpu.sync_copy(acc.at[pl.ds(wid * nw, nw)], buf.at[pl.ds(0, nw)])
        pltpu.sync_copy(buf.at[pl.ds(0, nw)], out_hbm.at[pl.ds(wid * nw, nw)])
        if with_gather:
            pltpu.sync_copy(src_hbm.at[pl.ds(wid * 2, 2)],
                            idx_v.at[pl.ds(0, 2)])
            for j in range(2):
                pltpu.async_copy(acc.at[idx_v.at[j]],
                                 buf.at[pl.ds(j * 128, 128)], sem).wait()
            pltpu.sync_copy(buf.at[pl.ds(0, _EPW)],
                            xs_hbm.at[pl.ds(wid * _EPW, _EPW)])

    return s


def _msg2_body(xs1_ref, xs2_ref, x1_ref, x2_ref, ea1_ref, ea2_ref,
               w01, b01, w11, b11, w2r1, b2r1, root1, bias1,
               w02, b02, w12, b12, w2r2, b2r2, root2, bias2,
               msg1_ref, msg2_ref, rt1_ref, rt2_ref, *, din, dout, first):
    for x_ref, root_ref, bias_ref, rt_ref in (
            (x1_ref, root1, bias1, rt1_ref), (x2_ref, root2, bias2, rt2_ref)):
        x = x_ref[...]
        if not first:
            x = jax.nn.relu(x)[:, :din]
        rt_ref[...] = x @ root_ref[...] + bias_ref[...]

    def body(b, _):
        sl = pl.ds(b * _BLK, _BLK)

        def load(xs_ref, ea_ref, w0, b0, w1, b1):
            xs = xs_ref[sl, :]
            if not first:
                xs = jax.nn.relu(xs)
            xs = xs[:, :din]
            h = jax.nn.relu(ea_ref[sl, :] @ w0[...] + b0[...])
            h = jax.nn.relu(h @ w1[...] + b1[...])
            return xs, h

        xs_1, h_1 = load(xs1_ref, ea1_ref, w01, b01, w11, b11)
        xs_2, h_2 = load(xs2_ref, ea2_ref, w02, b02, w12, b12)
        m1 = xs_1 @ b2r1[...]
        m2 = xs_2 @ b2r2[...]
        if din == 4:
            t1 = h_1 @ w2r1[...]
            t2 = h_2 @ w2r2[...]
            for i in range(din):
                m1 = m1 + xs_1[:, i:i + 1] * t1[:, i * dout:(i + 1) * dout]
                m2 = m2 + xs_2[:, i:i + 1] * t2[:, i * dout:(i + 1) * dout]
        else:
            def pchunk(h, xs, k0):
                hc = h[:, k0:k0 + _KC]
                hrep = jnp.broadcast_to(hc[:, :, None],
                                        (_BLK, _KC, din)).reshape(_BLK, _KC * din)
                xst = jnp.broadcast_to(xs[:, None, :],
                                       (_BLK, _KC, din)).reshape(_BLK, _KC * din)
                return hrep * xst

            for k0 in range(0, 64, _KC):
                kw = pl.ds(k0 * din, _KC * din)
                m1 = m1 + pchunk(h_1, xs_1, k0) @ w2r1[kw, :]
                m2 = m2 + pchunk(h_2, xs_2, k0) @ w2r2[kw, :]
        for m, ref in ((m1, msg1_ref), (m2, msg2_ref)):
            if dout < _W:
                m = jnp.concatenate(
                    [m, jnp.zeros((_BLK, _W - dout), jnp.float32)], axis=1)
            ref[sl, :] = m
        return 0

    jax.lax.fori_loop(0, _E // _BLK, body, 0)


def _edge_msg2(x1, x2, xs1, xs2, ea1, ea2, p1, p2, first):
    din, dout = p1['root'].shape

    def prep(p):
        w2r = p['w2'] if din == 4 else p['w2'].reshape(64 * din, dout)
        return (p['w0'], p['b0'].reshape(1, 64),
                p['w1'], p['b1'].reshape(1, 64),
                w2r, p['b2'].reshape(din, dout),
                jnp.pad(p['root'], ((0, 0), (0, _W - dout))),
                jnp.pad(p['bias'], (0, _W - dout)).reshape(1, _W))

    f = pl.pallas_call(
        functools.partial(_msg2_body, din=din, dout=dout, first=first),
        out_shape=(jax.ShapeDtypeStruct((_E, _W), jnp.float32),
                   jax.ShapeDtypeStruct((_E, _W), jnp.float32),
                   jax.ShapeDtypeStruct((_N, _W), jnp.float32),
                   jax.ShapeDtypeStruct((_N, _W), jnp.float32)),
    )
    return f(xs1, xs2, x1, x2, ea1, ea2, *prep(p1), *prep(p2))


def _head_body(x1_ref, x2_ref, iin_ref, iout_ref, l1w_ref, l1b_ref, l2w_ref,
               l2b_ref, l3w_ref, l3b_ref, ow_ref, ob_ref, out_ref):
    wl = (l1w_ref[...] @ l2w_ref[...]) @ l3w_ref[...]
    bl = (l1b_ref[...] @ l2w_ref[...] + l2b_ref[...]) @ l3w_ref[...] + l3b_ref[...]
    iota_gn = jax.lax.broadcasted_iota(jnp.int32, (_G, _N), 1)
    oh_in = (jnp.reshape(iin_ref[...], (_G, 1)) == iota_gn).astype(jnp.float32)
    oh_out = (jnp.reshape(iout_ref[...], (_G, 1)) == iota_gn).astype(jnp.float32)
    x1 = jax.nn.relu(x1_ref[...])[:, :64]
    x2 = jax.nn.relu(x2_ref[...])[:, :64]
    a_in = jnp.concatenate([oh_in @ x1, oh_in @ x2], axis=1) @ wl + bl
    a_out = jnp.concatenate([oh_out @ x1, oh_out @ x2], axis=1) @ wl + bl
    cat = jnp.concatenate([a_in, a_out], axis=1)
    out_ref[...] = jnp.sum(cat * ow_ref[...], axis=1, keepdims=True) + ob_ref[...]


def kernel(x, edge_index, edge_attr1, edge_attr2, batch, params):
    src2d = edge_index[0].reshape(_E // 128, 128)
    dst2d = edge_index[1].reshape(_E // 128, 128)
    counts = jnp.bincount(batch, length=_G)
    starts = (jnp.cumsum(counts) - counts).astype(jnp.int32)

    x0p = jnp.pad(x, ((0, 0), (0, _W - x.shape[1])))
    gather = _make_gather()
    scatter_g = _make_scatter(True)
    scatter_o = _make_scatter(False)

    names1 = ('c1a', 'c1b', 'c1c', 'c1d')
    names2 = ('c2a', 'c2b', 'c2c', 'c2d')
    x1 = x2 = x
    xs1 = gather(x0p, src2d)
    xs2 = gather(x0p, src2d)
    first = True
    for i, (n1, n2) in enumerate(zip(names1, names2)):
        msg1, msg2, rt1, rt2 = _edge_msg2(x1, x2, xs1, xs2, edge_attr1,
                                          edge_attr2, params[n1], params[n2],
                                          first)
        if i < 3:
            x1, xs1 = scatter_g(msg1, dst2d, rt1, src2d)
            x2, xs2 = scatter_g(msg2, dst2d, rt2, src2d)
        else:
            x1 = scatter_o(msg1, dst2d, rt1)
            x2 = scatter_o(msg2, dst2d, rt2)
        first = False

    head = pl.pallas_call(
        _head_body,
        out_shape=jax.ShapeDtypeStruct((_G, 1), jnp.float32),
    )
    return head(x1, x2, starts, starts + 1,
                params['lin1_w'], params['lin1_b'].reshape(1, 128),
                params['lin2_w'], params['lin2_b'].reshape(1, 64),
                params['lin3_w'], params['lin3_b'].reshape(1, 64),
                params['out_w'].reshape(1, 128), params['out_b'].reshape(1, 1))
